# baseline (speedup 1.0000x reference)
import jax
import jax.numpy as jnp
from jax.experimental import pallas as pl
from jax.experimental.pallas import tpu as pltpu

_NUM_CLASSES = 8192
_Z_DIM = 256
_N_CHUNKS = 8
_CHUNK = _NUM_CLASSES // _N_CHUNKS


def _copy_body(a_hbm, o_hbm, buf, in_sems, out_sems):
    for i in range(_N_CHUNKS):
        rows = pl.ds(i * _CHUNK, _CHUNK)
        pltpu.make_async_copy(a_hbm.at[rows, :], buf.at[i], in_sems.at[i]).start()
    for i in range(_N_CHUNKS):
        rows = pl.ds(i * _CHUNK, _CHUNK)
        pltpu.make_async_copy(a_hbm.at[rows, :], buf.at[i], in_sems.at[i]).wait()
        pltpu.make_async_copy(buf.at[i], o_hbm.at[rows, :], out_sems.at[i]).start()
    for i in range(_N_CHUNKS):
        rows = pl.ds(i * _CHUNK, _CHUNK)
        pltpu.make_async_copy(buf.at[i], o_hbm.at[rows, :], out_sems.at[i]).wait()


def kernel(_, anchor):
    return pl.pallas_call(
        _copy_body,
        in_specs=[pl.BlockSpec(memory_space=pl.ANY)],
        out_specs=pl.BlockSpec(memory_space=pl.ANY),
        out_shape=jax.ShapeDtypeStruct((_NUM_CLASSES, _Z_DIM), jnp.float32),
        scratch_shapes=[
            pltpu.VMEM((_N_CHUNKS, _CHUNK, _Z_DIM), jnp.float32),
            pltpu.SemaphoreType.DMA((_N_CHUNKS,)),
            pltpu.SemaphoreType.DMA((_N_CHUNKS,)),
        ],
    )(anchor)
